# baseline (speedup 1.0000x reference)
import jax
import jax.numpy as jnp
from jax import lax
from jax.experimental import pallas as pl
from jax.experimental.pallas import tpu as pltpu
from jax.experimental.pallas import tpu_sc as plsc

N = 10000
E = 320000
D_IN = 128
HID = 64
OUT = 8
NPOW = 5
SCALING = 2.0

NC = 2
NS = 16
NW = NC * NS
N_PAD = 10240
ROWS_PER_TILE = N_PAD // NS
E_PAD = 327680
EROWS = E_PAD // 128
RW = EROWS // NW
CHR = 8
NCHUNK = RW // CHR
DUMP = N_PAD - 2

_i32 = jnp.int32
_f32 = jnp.float32


def _iota16():
    return lax.iota(_i32, 16)


def _zc16():
    return jnp.zeros((16,), _i32)


def _stage_a_body(x_ref, wp1_ref, bp1_ref, wp2_ref, bp2_ref, wp3_ref, bp3_ref,
                  p16_ref, w0cat_ref, w00_ref, b00_ref,
                  at_ref, bt_ref, q_ref, p0_ref):
    x = x_ref[...]
    h = jnp.maximum(jnp.dot(x, wp1_ref[...], preferred_element_type=_f32)
                    + bp1_ref[...], 0.0)
    h = jnp.maximum(jnp.dot(h, wp2_ref[...], preferred_element_type=_f32)
                    + bp2_ref[...], 0.0)
    logits = (jnp.dot(h, wp3_ref[...], preferred_element_type=_f32)
              + bp3_ref[...])[:, :OUT]
    a16 = jnp.concatenate([logits, jnp.zeros_like(logits)], axis=1)
    at_ref[...] = a16
    bt_ref[...] = jnp.dot(a16, p16_ref[...], preferred_element_type=_f32)
    q_ref[...] = jnp.dot(x, w0cat_ref[...], preferred_element_type=_f32)
    p0_ref[...] = jnp.dot(x, w00_ref[...], preferred_element_type=_f32) + b00_ref[...]


def _stage_a(xP, Wp1, bp1, Wp2, bp2, Wp3, bp3, P16, W0cat, W00, b00):
    blk = 640
    grid = N_PAD // blk
    full = lambda s: pl.BlockSpec(s, lambda i: (0, 0))
    return pl.pallas_call(
        _stage_a_body,
        grid=(grid,),
        in_specs=[
            pl.BlockSpec((blk, D_IN), lambda i: (i, 0)),
            full((D_IN, 512)), full((1, 512)),
            full((512, 64)), full((1, 64)),
            full((64, 128)), full((1, 128)),
            full((16, 16)),
            full((D_IN, 256)),
            full((D_IN, 64)), full((1, 64)),
        ],
        out_specs=[
            pl.BlockSpec((blk, 16), lambda i: (i, 0)),
            pl.BlockSpec((blk, 16), lambda i: (i, 0)),
            pl.BlockSpec((blk, 256), lambda i: (i, 0)),
            pl.BlockSpec((blk, 64), lambda i: (i, 0)),
        ],
        out_shape=[
            jax.ShapeDtypeStruct((N_PAD, 16), _f32),
            jax.ShapeDtypeStruct((N_PAD, 16), _f32),
            jax.ShapeDtypeStruct((N_PAD, 256), _f32),
            jax.ShapeDtypeStruct((N_PAD, 64), _f32),
        ],
    )(xP, Wp1, bp1, Wp2, bp2, Wp3, bp3, P16, W0cat, W00, b00)


def _stage_b_body(at_hbm, bt_hbm, src_hbm, dst_hbm,
                  raw_hbm, stats_hbm, degp_hbm,
                  sbuf, dbuf, arows, brows, prodf, valbuf, ewbuf, statbuf,
                  zbuf, degacc, sem):
    c = lax.axis_index("c")
    s = lax.axis_index("s")
    w = c * NS + s
    it16 = _iota16()
    z16f = jnp.zeros((16,), _f32)
    lane1 = jnp.where(it16 == 1, 1.0, 0.0).astype(_f32)

    r0 = s * ROWS_PER_TILE

    def zrow(rr, _):
        zbuf[rr, :] = z16f
        return 0
    lax.fori_loop(0, 128, zrow, 0)
    for zz in range(ROWS_PER_TILE // 128):
        pltpu.sync_copy(zbuf, degacc.at[pl.ds(r0 + zz * 128, 128)])
    plsc.subcore_barrier()

    def chunk_body(cc, carry):
        rb = w * RW + cc * CHR
        e0 = rb * 128
        pltpu.sync_copy(src_hbm.at[pl.ds(e0, CHR * 128)], sbuf)
        pltpu.sync_copy(dst_hbm.at[pl.ds(rb, CHR)], dbuf)
        cps = []
        for jj in range(CHR):
            cps.append(pltpu.async_copy(
                at_hbm.at[sbuf.at[pl.ds(jj * 128, 128)]],
                arows.at[pl.ds(jj * 128, 128)], sem))
            cps.append(pltpu.async_copy(
                bt_hbm.at[dbuf.at[jj]], brows.at[pl.ds(jj * 128, 128)], sem))
        for cp in cps:
            cp.wait()

        def prod_body(le, _):
            prodf[pl.ds(le * 16, 16)] = arows[le, :] * brows[le, :]
            return 0
        lax.fori_loop(0, CHR * 128, prod_body, 0)

        def group_body(g, carry2):
            ssum2, ssq2 = carry2
            acc = jnp.zeros((16,), _f32)
            for cdim in range(OUT):
                a_c = plsc.load_gather(prodf, [it16 * 16 + (g * 256 + cdim)])
                acc = acc + a_c
            ewbuf[pl.ds(g * 16, 16)] = acc
            base = rb * 128 + g * 16
            m = (it16 + base) < E
            ewm = jnp.where(m, acc, 0.0)
            return ssum2 + ewm, ssq2 + ewm * ewm

        carry = lax.fori_loop(0, CHR * 8, group_body, carry)

        def val_body(le, _):
            sp = plsc.load_gather(ewbuf, [_zc16() + le])
            valbuf[le, :] = jnp.where(it16 == 0, sp, lane1)
            return 0
        lax.fori_loop(0, CHR * 128, val_body, 0)

        pltpu.sync_copy(ewbuf, raw_hbm.at[pl.ds(e0, CHR * 128)])
        for jj in range(CHR):
            pltpu.sync_copy(valbuf.at[pl.ds(jj * 128, 128)],
                            degacc.at[dbuf.at[jj]], add=True)
        return carry

    ssum, ssq = lax.fori_loop(0, NCHUNK, chunk_body,
                              (jnp.zeros((16,), _f32), jnp.zeros((16,), _f32)))

    statbuf[pl.ds(0, 16)] = ssum
    statbuf[pl.ds(16, 16)] = ssq
    pltpu.sync_copy(statbuf, stats_hbm.at[w])

    plsc.subcore_barrier()
    pltpu.sync_copy(degacc.at[pl.ds(r0, ROWS_PER_TILE)],
                    degp_hbm.at[c, pl.ds(r0, ROWS_PER_TILE)])


def _stage_b(At, Bt, srcF, dstR):
    mesh = plsc.VectorSubcoreMesh(core_axis_name="c", subcore_axis_name="s")
    f = pl.kernel(
        _stage_b_body,
        out_type=[
            jax.ShapeDtypeStruct((E_PAD,), _f32),
            jax.ShapeDtypeStruct((NW, 32), _f32),
            jax.ShapeDtypeStruct((NC, N_PAD, 16), _f32),
        ],
        mesh=mesh,
        compiler_params=pltpu.CompilerParams(needs_layout_passes=False, use_tc_tiling_on_sc=False),
        scratch_types=[
            pltpu.VMEM((CHR * 128,), _i32),
            pltpu.VMEM((CHR, 128), _i32),
            pltpu.VMEM((CHR * 128, 16), _f32),
            pltpu.VMEM((CHR * 128, 16), _f32),
            pltpu.VMEM((CHR * 128 * 16,), _f32),
            pltpu.VMEM((CHR * 128, 16), _f32),
            pltpu.VMEM((CHR * 128,), _f32),
            pltpu.VMEM((32,), _f32),
            pltpu.VMEM((128, 16), _f32),
            pltpu.VMEM_SHARED((N_PAD, 16), _f32),
            pltpu.SemaphoreType.DMA,
        ],
    )
    return f(At, Bt, srcF, dstR)


def _stage_c2_body(stats_ref, degp_ref, ms_ref, dinv_ref, sn_ref):
    stats = stats_ref[...]
    ssum = jnp.sum(stats[:, 0:16])
    ssq = jnp.sum(stats[:, 16:32])
    mean = ssum / E
    var = (ssq - E * mean * mean) / (E - 1)
    scale = jnp.sqrt(0.0001 / var)

    degp = degp_ref[...]
    rawsum = degp[0, :, 0:1] + degp[1, :, 0:1]
    cnt = degp[0, :, 1:2] + degp[1, :, 1:2]
    deg = scale * rawsum + (1.0 - scale * mean) * cnt + 1.0
    dinv = jnp.where(deg > 0, lax.rsqrt(deg), 0.0)
    rows = lax.broadcasted_iota(_i32, (N_PAD, 1), 0)
    dinv = jnp.where(rows < N, dinv, 0.0)
    sn = dinv * dinv
    dinv_ref[...] = jnp.broadcast_to(dinv, (N_PAD, 8))
    sn_ref[...] = jnp.broadcast_to(sn, (N_PAD, 8))
    col = lax.broadcasted_iota(_i32, (8, 128), 1)
    ms_ref[...] = jnp.where(col == 0, mean, jnp.where(col == 1, scale, 0.0))


def _stage_c2(statsP, degP):
    return pl.pallas_call(
        _stage_c2_body,
        out_shape=[
            jax.ShapeDtypeStruct((8, 128), _f32),
            jax.ShapeDtypeStruct((N_PAD, 8), _f32),
            jax.ShapeDtypeStruct((N_PAD, 8), _f32),
        ],
    )(statsP, degP)


def _stage_c3_body(dinv_hbm, ms_hbm, src_hbm, dst_hbm, raw_hbm,
                   norm_hbm, dinvbuf, msbuf, sbuf, dbuf, rbuf, nbuf, sem):
    c = lax.axis_index("c")
    s = lax.axis_index("s")
    w = c * NS + s
    pltpu.sync_copy(dinv_hbm, dinvbuf)
    pltpu.sync_copy(ms_hbm, msbuf)
    mean = plsc.load_gather(msbuf, [_zc16()])
    scale = plsc.load_gather(msbuf, [_zc16() + 1])

    def chunk(cc, _):
        e0 = (w * RW + cc * CHR) * 128
        pltpu.sync_copy(src_hbm.at[pl.ds(e0, CHR * 128)], sbuf)
        pltpu.sync_copy(dst_hbm.at[pl.ds(e0, CHR * 128)], dbuf)
        pltpu.sync_copy(raw_hbm.at[pl.ds(e0, CHR * 128)], rbuf)

        def group(g, _2):
            sv = plsc.load_gather(dinvbuf, [sbuf[pl.ds(g * 16, 16)]])
            dv = plsc.load_gather(dinvbuf, [dbuf[pl.ds(g * 16, 16)]])
            raw = rbuf[pl.ds(g * 16, 16)]
            ewp = (raw - mean) * scale + 1.0
            nbuf[pl.ds(g * 16, 16)] = sv * ewp * dv
            return 0
        lax.fori_loop(0, CHR * 8, group, 0)
        pltpu.sync_copy(nbuf, norm_hbm.at[pl.ds(e0, CHR * 128)])
        return 0

    lax.fori_loop(0, NCHUNK, chunk, 0)


def _stage_c3(dinvA, msvec, srcF, dstF, rawF):
    mesh = plsc.VectorSubcoreMesh(core_axis_name="c", subcore_axis_name="s")
    f = pl.kernel(
        _stage_c3_body,
        out_type=jax.ShapeDtypeStruct((E_PAD,), _f32),
        mesh=mesh,
        compiler_params=pltpu.CompilerParams(needs_layout_passes=False, use_tc_tiling_on_sc=False),
        scratch_types=[
            pltpu.VMEM((N_PAD,), _f32),
            pltpu.VMEM((16,), _f32),
            pltpu.VMEM((CHR * 128,), _i32),
            pltpu.VMEM((CHR * 128,), _i32),
            pltpu.VMEM((CHR * 128,), _f32),
            pltpu.VMEM((CHR * 128,), _f32),
            pltpu.SemaphoreType.DMA,
        ],
    )
    return f(dinvA, msvec, srcF, dstF, rawF)


SCH = 8
SNCH = RW // SCH
PW = 32


def _scatter_body(v_hbm, src_hbm, dst_hbm, norm_hbm, tok_hbm,
                  pout_hbm, tokout_hbm, sbufA, dbufA, nbufA, rows0, rows1, acc,
                  tokbuf, semg0, semg1, sems0, sems1):
    c = lax.axis_index("c")
    s = lax.axis_index("s")
    w = c * NS + s
    r0 = s * ROWS_PER_TILE
    eb = w * RW * 128
    rb = w * RW

    pltpu.sync_copy(tok_hbm, tokbuf)

    @pl.when(w == 0)
    def _():
        pltpu.sync_copy(tokbuf, tokout_hbm)

    pltpu.sync_copy(src_hbm.at[pl.ds(eb, RW * 128)], sbufA)
    pltpu.sync_copy(dst_hbm.at[pl.ds(rb, RW)], dbufA)
    pltpu.sync_copy(norm_hbm.at[pl.ds(eb, RW * 128)], nbufA)

    z16f = jnp.zeros((16,), _f32)

    def zrow(rr, _):
        for k in range(PW // 16):
            rows0[rr, pl.ds(k * 16, 16)] = z16f
        return 0
    lax.fori_loop(0, 128, zrow, 0)
    for zz in range(ROWS_PER_TILE // 128):
        pltpu.sync_copy(rows0.at[pl.ds(0, 128)],
                        acc.at[pl.ds(r0 + zz * 128, 128)])
    plsc.subcore_barrier()

    rows = (rows0, rows1)
    semg = (semg0, semg1)
    sems = (sems0, sems1)

    def fire_gather(cc, b):
        for jj in range(SCH):
            pltpu.async_copy(
                v_hbm.at[sbufA.at[pl.ds((cc * SCH + jj) * 128, 128)]],
                rows[b].at[pl.ds(jj * 128, 128)], semg[b])

    def drain_gather(cc, b):
        for jj in range(SCH):
            pltpu.make_async_copy(
                v_hbm.at[sbufA.at[pl.ds((cc * SCH + jj) * 128, 128)]],
                rows[b].at[pl.ds(jj * 128, 128)], semg[b]).wait()

    def fire_scatter(cc, b):
        for jj in range(SCH):
            pltpu.async_copy(
                rows[b].at[pl.ds(jj * 128, 128)],
                acc.at[dbufA.at[cc * SCH + jj]], sems[b], add=True)

    def drain_scatter(cc, b):
        for jj in range(SCH):
            pltpu.make_async_copy(
                rows[b].at[pl.ds(jj * 128, 128)],
                acc.at[dbufA.at[cc * SCH + jj]], sems[b]).wait()

    def scale(cc, b):
        cb = cc * SCH * 128

        def group(g, _2):
            for i in range(16):
                sp = plsc.load_gather(nbufA, [_zc16() + (cb + g * 16 + i)])
                le = g * 16 + i
                for k in range(PW // 16):
                    rows[b][le, pl.ds(k * 16, 16)] = (
                        rows[b][le, pl.ds(k * 16, 16)] * sp)
            return 0
        lax.fori_loop(0, SCH * 8, group, 0)

    def proc(cc, b, first):
        drain_gather(cc, b)
        scale(cc, b)
        fire_scatter(cc, b)
        if not first:
            drain_scatter(cc - 1, 1 - b)
        nxt = jnp.where(cc + 1 < SNCH, cc + 1, 0)
        fire_gather(nxt, 1 - b)

    fire_gather(0, 0)
    proc(jnp.int32(0), 0, True)

    def step(cc2, _):
        proc(cc2 * 2 + 1, 1, False)
        proc(cc2 * 2 + 2, 0, False)
        return 0
    lax.fori_loop(0, (SNCH - 2) // 2, step, 0)

    proc(jnp.int32(SNCH - 1), 1, False)
    drain_scatter(SNCH - 1, 1)
    drain_gather(0, 0)

    plsc.subcore_barrier()
    pltpu.sync_copy(acc.at[pl.ds(r0, ROWS_PER_TILE)],
                    pout_hbm.at[c, pl.ds(r0, ROWS_PER_TILE)])


def _sc_scatter(v, srcF, dstR, normF, tok):
    mesh = plsc.VectorSubcoreMesh(core_axis_name="c", subcore_axis_name="s")
    f = pl.kernel(
        _scatter_body,
        out_type=[pltpu.HBM((NC, N_PAD, PW), _f32),
                  pltpu.HBM((16,), _f32)],
        mesh=mesh,
        compiler_params=pltpu.CompilerParams(needs_layout_passes=False, use_tc_tiling_on_sc=False),
        scratch_types=[
            pltpu.VMEM((RW * 128,), _i32),
            pltpu.VMEM((RW, 128), _i32),
            pltpu.VMEM((RW * 128,), _f32),
            pltpu.VMEM((SCH * 128, PW), _f32),
            pltpu.VMEM((SCH * 128, PW), _f32),
            pltpu.VMEM_SHARED((N_PAD, PW), _f32),
            pltpu.VMEM((16,), _f32),
            pltpu.SemaphoreType.DMA,
            pltpu.SemaphoreType.DMA,
            pltpu.SemaphoreType.DMA,
            pltpu.SemaphoreType.DMA,
        ],
    )
    return f(v, srcF, dstR, normF, tok)


def _combine_body(p_ref, v_ref, sn_ref, o_ref):
    sn = sn_ref[...][:, 0:1]
    o_ref[...] = p_ref[0] + p_ref[1] + sn * v_ref[...]


def _tc_combine(p, v, sn8):
    blk = 1280
    return pl.pallas_call(
        _combine_body,
        grid=(N_PAD // blk,),
        in_specs=[
            pl.BlockSpec((NC, blk, PW), lambda i: (0, i, 0)),
            pl.BlockSpec((blk, PW), lambda i: (i, 0)),
            pl.BlockSpec((blk, 8), lambda i: (i, 0)),
        ],
        out_specs=pl.BlockSpec((blk, PW), lambda i: (i, 0)),
        out_shape=jax.ShapeDtypeStruct((N_PAD, PW), _f32),
    )(p, v, sn8)


def _stage_e_body(p0_ref, c1_ref, c2_ref, c3_ref, c4_ref, b0_ref,
                  w1cat_ref, w10_ref, b10_ref, r_ref, r0_ref):
    b0 = b0_ref[...]
    h1 = jnp.concatenate([
        jnp.maximum(p0_ref[...], 0.0),
        jnp.maximum(c1_ref[...] + b0[1:2, :], 0.0),
        jnp.maximum(c2_ref[...] + b0[2:3, :], 0.0),
        jnp.maximum(c3_ref[...] + b0[3:4, :], 0.0),
        jnp.maximum(c4_ref[...] + b0[4:5, :], 0.0),
    ], axis=1)
    r_ref[...] = jnp.dot(h1, w1cat_ref[...], preferred_element_type=_f32)
    r0_ref[...] = (jnp.dot(h1, w10_ref[...], preferred_element_type=_f32)
                   + b10_ref[...])


def _stage_e(p0, c1, c2, c3, c4, b0p, W1cat, W10, b10):
    blk = 640
    full = lambda s: pl.BlockSpec(s, lambda i: (0, 0))
    row = lambda d: pl.BlockSpec((blk, d), lambda i: (i, 0))
    return pl.pallas_call(
        _stage_e_body,
        grid=(N_PAD // blk,),
        in_specs=[row(64), row(64), row(64), row(64), row(64),
                  full((8, 64)),
                  full((320, 256)), full((320, 64)), full((1, 64))],
        out_specs=[row(256), row(64)],
        out_shape=[
            jax.ShapeDtypeStruct((N_PAD, 256), _f32),
            jax.ShapeDtypeStruct((N_PAD, 64), _f32),
        ],
    )(p0, c1, c2, c3, c4, b0p, W1cat, W10, b10)


def _stage_f_body(r0_ref, d1_ref, d2_ref, d3_ref, d4_ref, b1_ref,
                  wf_ref, bf_ref, o_ref):
    b1 = b1_ref[...]
    h2 = jnp.concatenate([
        jnp.maximum(r0_ref[...], 0.0),
        jnp.maximum(d1_ref[...] + b1[1:2, :], 0.0),
        jnp.maximum(d2_ref[...] + b1[2:3, :], 0.0),
        jnp.maximum(d3_ref[...] + b1[3:4, :], 0.0),
        jnp.maximum(d4_ref[...] + b1[4:5, :], 0.0),
    ], axis=1)
    o_ref[...] = (jnp.dot(h2, wf_ref[...], preferred_element_type=_f32)
                  + bf_ref[...])


def _stage_f(r0, d1, d2, d3, d4, b1p, Wf, bf):
    blk = 640
    full = lambda s: pl.BlockSpec(s, lambda i: (0, 0))
    row = lambda d: pl.BlockSpec((blk, d), lambda i: (i, 0))
    return pl.pallas_call(
        _stage_f_body,
        grid=(N_PAD // blk,),
        in_specs=[row(64), row(64), row(64), row(64), row(64),
                  full((8, 64)), full((320, OUT)), full((1, OUT))],
        out_specs=row(OUT),
        out_shape=jax.ShapeDtypeStruct((N_PAD, OUT), _f32),
    )(r0, d1, d2, d3, d4, b1p, Wf, bf)


def kernel(x, edge_index, lins0_W, lins0_b, lins1_W, lins1_b, Wf, bf,
           Wp1, bp1, Wp2, bp2, Wp3, bp3, parsing0):
    src = edge_index[0]
    dst = edge_index[1]
    pad = E_PAD - E
    srcF = jnp.concatenate([src, jnp.zeros((pad,), _i32)])
    dstF = jnp.concatenate([dst, jnp.full((pad,), DUMP, _i32)])
    dstR = dstF.reshape(EROWS, 128)
    xP = jnp.zeros((N_PAD, D_IN), _f32).at[:N].set(x)

    P16 = jnp.zeros((16, 16), _f32).at[:OUT, :OUT].set(
        jax.nn.relu(SCALING * parsing0))
    W0cat = jnp.concatenate([lins0_W[j] for j in range(1, NPOW)], axis=1)
    W1cat = jnp.concatenate([lins1_W[j] for j in range(1, NPOW)], axis=1)
    b0p = jnp.zeros((8, 64), _f32).at[:NPOW].set(lins0_b)
    b1p = jnp.zeros((8, 64), _f32).at[:NPOW].set(lins1_b)
    Wp3p = jnp.zeros((64, 128), _f32).at[:, :OUT].set(Wp3)
    bp3p = jnp.zeros((128,), _f32).at[:OUT].set(bp3)

    At, Bt, q, p0 = _stage_a(xP, Wp1, bp1[None], Wp2, bp2[None],
                             Wp3p, bp3p[None], P16, W0cat,
                             lins0_W[0], lins0_b[0][None])

    rawF, statsP, degP = _stage_b(At, Bt, srcF, dstR)
    msarr, dinv8, sn8 = _stage_c2(statsP, degP)
    dinvA = dinv8[:, 0]
    msvec = msarr[0, :16]
    normF = _stage_c3(dinvA, msvec, srcF, dstF, rawF)

    def S(v, tok):
        p, tok2 = _sc_scatter(v, srcF, dstR, normF, tok)
        return _tc_combine(p, v, sn8), tok2

    def run_chains(chains, tok):
        outs = []
        cur = list(chains)
        for step in range(len(cur)):
            for j in range(step, len(cur)):
                cur[j], tok = S(cur[j], tok)
            outs.append(cur[step])
        return outs, tok

    def run_layer(m, tok):
        halves = [m[:, 32 * t:32 * (t + 1)] for t in range(8)]
        outs = [None] * 8
        for step in range(4):
            for t in range(2 * step, 8):
                halves[t], _unused = S(halves[t], tok)
            outs[2 * step] = halves[2 * step]
            outs[2 * step + 1] = halves[2 * step + 1]
        full = [jnp.concatenate([outs[2 * jj], outs[2 * jj + 1]], axis=1)
                for jj in range(4)]
        return full, tok

    tok = msvec
    (c1, c2, c3, c4), tok = run_layer(q, tok)
    r, r0 = _stage_e(p0, c1, c2, c3, c4, b0p, W1cat, lins1_W[0],
                     lins1_b[0][None])
    (d1, d2, d3, d4), tok = run_layer(r, tok)
    out = _stage_f(r0, d1, d2, d3, d4, b1p, Wf, bf[None])
    return out[:N]


# speedup vs baseline: 1.0688x; 1.0688x over previous
import jax
import jax.numpy as jnp
from jax import lax
from jax.experimental import pallas as pl
from jax.experimental.pallas import tpu as pltpu
from jax.experimental.pallas import tpu_sc as plsc

N = 10000
E = 320000
D_IN = 128
HID = 64
OUT = 8
NPOW = 5
SCALING = 2.0

NC = 2
NS = 16
NW = NC * NS
N_PAD = 10240
ROWS_PER_TILE = N_PAD // NS
E_PAD = 327680
EROWS = E_PAD // 128
RW = EROWS // NW
CHR = 8
NCHUNK = RW // CHR
DUMP = N_PAD - 2

_i32 = jnp.int32
_f32 = jnp.float32


def _iota16():
    return lax.iota(_i32, 16)


def _zc16():
    return jnp.zeros((16,), _i32)


def _stage_a_body(x_ref, wp1_ref, bp1_ref, wp2_ref, bp2_ref, wp3_ref, bp3_ref,
                  p16_ref, w0cat_ref, w00_ref, b00_ref,
                  at_ref, bt_ref, q_ref, p0_ref):
    x = x_ref[...]
    h = jnp.maximum(jnp.dot(x, wp1_ref[...], preferred_element_type=_f32)
                    + bp1_ref[...], 0.0)
    h = jnp.maximum(jnp.dot(h, wp2_ref[...], preferred_element_type=_f32)
                    + bp2_ref[...], 0.0)
    logits = (jnp.dot(h, wp3_ref[...], preferred_element_type=_f32)
              + bp3_ref[...])[:, :OUT]
    a16 = jnp.concatenate([logits, jnp.zeros_like(logits)], axis=1)
    at_ref[...] = a16
    bt_ref[...] = jnp.dot(a16, p16_ref[...], preferred_element_type=_f32)
    q_ref[...] = jnp.dot(x, w0cat_ref[...], preferred_element_type=_f32)
    p0_ref[...] = jnp.dot(x, w00_ref[...], preferred_element_type=_f32) + b00_ref[...]


def _stage_a(xP, Wp1, bp1, Wp2, bp2, Wp3, bp3, P16, W0cat, W00, b00):
    blk = 640
    grid = N_PAD // blk
    full = lambda s: pl.BlockSpec(s, lambda i: (0, 0))
    return pl.pallas_call(
        _stage_a_body,
        grid=(grid,),
        in_specs=[
            pl.BlockSpec((blk, D_IN), lambda i: (i, 0)),
            full((D_IN, 512)), full((1, 512)),
            full((512, 64)), full((1, 64)),
            full((64, 128)), full((1, 128)),
            full((16, 16)),
            full((D_IN, 256)),
            full((D_IN, 64)), full((1, 64)),
        ],
        out_specs=[
            pl.BlockSpec((blk, 16), lambda i: (i, 0)),
            pl.BlockSpec((blk, 16), lambda i: (i, 0)),
            pl.BlockSpec((blk, 256), lambda i: (i, 0)),
            pl.BlockSpec((blk, 64), lambda i: (i, 0)),
        ],
        out_shape=[
            jax.ShapeDtypeStruct((N_PAD, 16), _f32),
            jax.ShapeDtypeStruct((N_PAD, 16), _f32),
            jax.ShapeDtypeStruct((N_PAD, 256), _f32),
            jax.ShapeDtypeStruct((N_PAD, 64), _f32),
        ],
    )(xP, Wp1, bp1, Wp2, bp2, Wp3, bp3, P16, W0cat, W00, b00)


def _stage_b_body(at_hbm, bt_hbm, src_hbm, dst_hbm,
                  raw_hbm, stats_hbm, degp_hbm,
                  sbuf, dbuf, arows, brows, prodf, valbuf, ewbuf, statbuf,
                  zbuf, degacc, sem):
    c = lax.axis_index("c")
    s = lax.axis_index("s")
    w = c * NS + s
    it16 = _iota16()
    z16f = jnp.zeros((16,), _f32)
    lane1 = jnp.where(it16 == 1, 1.0, 0.0).astype(_f32)

    r0 = s * ROWS_PER_TILE

    def zrow(rr, _):
        zbuf[rr, :] = z16f
        return 0
    lax.fori_loop(0, 128, zrow, 0)
    for zz in range(ROWS_PER_TILE // 128):
        pltpu.sync_copy(zbuf, degacc.at[pl.ds(r0 + zz * 128, 128)])
    plsc.subcore_barrier()

    def chunk_body(cc, carry):
        rb = w * RW + cc * CHR
        e0 = rb * 128
        pltpu.sync_copy(src_hbm.at[pl.ds(e0, CHR * 128)], sbuf)
        pltpu.sync_copy(dst_hbm.at[pl.ds(rb, CHR)], dbuf)
        cps = []
        for jj in range(CHR):
            cps.append(pltpu.async_copy(
                at_hbm.at[sbuf.at[pl.ds(jj * 128, 128)]],
                arows.at[pl.ds(jj * 128, 128)], sem))
            cps.append(pltpu.async_copy(
                bt_hbm.at[dbuf.at[jj]], brows.at[pl.ds(jj * 128, 128)], sem))
        for cp in cps:
            cp.wait()

        def prod_body(le, _):
            prodf[pl.ds(le * 16, 16)] = arows[le, :] * brows[le, :]
            return 0
        lax.fori_loop(0, CHR * 128, prod_body, 0)

        def group_body(g, carry2):
            ssum2, ssq2 = carry2
            acc = jnp.zeros((16,), _f32)
            for cdim in range(OUT):
                a_c = plsc.load_gather(prodf, [it16 * 16 + (g * 256 + cdim)])
                acc = acc + a_c
            ewbuf[pl.ds(g * 16, 16)] = acc
            base = rb * 128 + g * 16
            m = (it16 + base) < E
            ewm = jnp.where(m, acc, 0.0)
            return ssum2 + ewm, ssq2 + ewm * ewm

        carry = lax.fori_loop(0, CHR * 8, group_body, carry)

        def val_body(le, _):
            sp = plsc.load_gather(ewbuf, [_zc16() + le])
            valbuf[le, :] = jnp.where(it16 == 0, sp, lane1)
            return 0
        lax.fori_loop(0, CHR * 128, val_body, 0)

        pltpu.sync_copy(ewbuf, raw_hbm.at[pl.ds(e0, CHR * 128)])
        for jj in range(CHR):
            pltpu.sync_copy(valbuf.at[pl.ds(jj * 128, 128)],
                            degacc.at[dbuf.at[jj]], add=True)
        return carry

    ssum, ssq = lax.fori_loop(0, NCHUNK, chunk_body,
                              (jnp.zeros((16,), _f32), jnp.zeros((16,), _f32)))

    statbuf[pl.ds(0, 16)] = ssum
    statbuf[pl.ds(16, 16)] = ssq
    pltpu.sync_copy(statbuf, stats_hbm.at[w])

    plsc.subcore_barrier()
    pltpu.sync_copy(degacc.at[pl.ds(r0, ROWS_PER_TILE)],
                    degp_hbm.at[c, pl.ds(r0, ROWS_PER_TILE)])


def _stage_b(At, Bt, srcF, dstR):
    mesh = plsc.VectorSubcoreMesh(core_axis_name="c", subcore_axis_name="s")
    f = pl.kernel(
        _stage_b_body,
        out_type=[
            jax.ShapeDtypeStruct((E_PAD,), _f32),
            jax.ShapeDtypeStruct((NW, 32), _f32),
            jax.ShapeDtypeStruct((NC, N_PAD, 16), _f32),
        ],
        mesh=mesh,
        compiler_params=pltpu.CompilerParams(needs_layout_passes=False, use_tc_tiling_on_sc=False),
        scratch_types=[
            pltpu.VMEM((CHR * 128,), _i32),
            pltpu.VMEM((CHR, 128), _i32),
            pltpu.VMEM((CHR * 128, 16), _f32),
            pltpu.VMEM((CHR * 128, 16), _f32),
            pltpu.VMEM((CHR * 128 * 16,), _f32),
            pltpu.VMEM((CHR * 128, 16), _f32),
            pltpu.VMEM((CHR * 128,), _f32),
            pltpu.VMEM((32,), _f32),
            pltpu.VMEM((128, 16), _f32),
            pltpu.VMEM_SHARED((N_PAD, 16), _f32),
            pltpu.SemaphoreType.DMA,
        ],
    )
    return f(At, Bt, srcF, dstR)


def _stage_c2_body(stats_ref, degp_ref, ms_ref, dinv_ref, sn_ref):
    stats = stats_ref[...]
    ssum = jnp.sum(stats[:, 0:16])
    ssq = jnp.sum(stats[:, 16:32])
    mean = ssum / E
    var = (ssq - E * mean * mean) / (E - 1)
    scale = jnp.sqrt(0.0001 / var)

    degp = degp_ref[...]
    rawsum = degp[0, :, 0:1] + degp[1, :, 0:1]
    cnt = degp[0, :, 1:2] + degp[1, :, 1:2]
    deg = scale * rawsum + (1.0 - scale * mean) * cnt + 1.0
    dinv = jnp.where(deg > 0, lax.rsqrt(deg), 0.0)
    rows = lax.broadcasted_iota(_i32, (N_PAD, 1), 0)
    dinv = jnp.where(rows < N, dinv, 0.0)
    sn = dinv * dinv
    dinv_ref[...] = jnp.broadcast_to(dinv, (N_PAD, 8))
    sn_ref[...] = jnp.broadcast_to(sn, (N_PAD, 8))
    col = lax.broadcasted_iota(_i32, (8, 128), 1)
    ms_ref[...] = jnp.where(col == 0, mean, jnp.where(col == 1, scale, 0.0))


def _stage_c2(statsP, degP):
    return pl.pallas_call(
        _stage_c2_body,
        out_shape=[
            jax.ShapeDtypeStruct((8, 128), _f32),
            jax.ShapeDtypeStruct((N_PAD, 8), _f32),
            jax.ShapeDtypeStruct((N_PAD, 8), _f32),
        ],
    )(statsP, degP)


def _stage_c3_body(dinv_hbm, ms_hbm, src_hbm, dst_hbm, raw_hbm,
                   norm_hbm, dinvbuf, msbuf, sbuf, dbuf, rbuf, nbuf, sem):
    c = lax.axis_index("c")
    s = lax.axis_index("s")
    w = c * NS + s
    pltpu.sync_copy(dinv_hbm, dinvbuf)
    pltpu.sync_copy(ms_hbm, msbuf)
    mean = plsc.load_gather(msbuf, [_zc16()])
    scale = plsc.load_gather(msbuf, [_zc16() + 1])

    def chunk(cc, _):
        e0 = (w * RW + cc * CHR) * 128
        pltpu.sync_copy(src_hbm.at[pl.ds(e0, CHR * 128)], sbuf)
        pltpu.sync_copy(dst_hbm.at[pl.ds(e0, CHR * 128)], dbuf)
        pltpu.sync_copy(raw_hbm.at[pl.ds(e0, CHR * 128)], rbuf)

        def group(g, _2):
            sv = plsc.load_gather(dinvbuf, [sbuf[pl.ds(g * 16, 16)]])
            dv = plsc.load_gather(dinvbuf, [dbuf[pl.ds(g * 16, 16)]])
            raw = rbuf[pl.ds(g * 16, 16)]
            ewp = (raw - mean) * scale + 1.0
            nbuf[pl.ds(g * 16, 16)] = sv * ewp * dv
            return 0
        lax.fori_loop(0, CHR * 8, group, 0)
        pltpu.sync_copy(nbuf, norm_hbm.at[pl.ds(e0, CHR * 128)])
        return 0

    lax.fori_loop(0, NCHUNK, chunk, 0)


def _stage_c3(dinvA, msvec, srcF, dstF, rawF):
    mesh = plsc.VectorSubcoreMesh(core_axis_name="c", subcore_axis_name="s")
    f = pl.kernel(
        _stage_c3_body,
        out_type=jax.ShapeDtypeStruct((E_PAD,), _f32),
        mesh=mesh,
        compiler_params=pltpu.CompilerParams(needs_layout_passes=False, use_tc_tiling_on_sc=False),
        scratch_types=[
            pltpu.VMEM((N_PAD,), _f32),
            pltpu.VMEM((16,), _f32),
            pltpu.VMEM((CHR * 128,), _i32),
            pltpu.VMEM((CHR * 128,), _i32),
            pltpu.VMEM((CHR * 128,), _f32),
            pltpu.VMEM((CHR * 128,), _f32),
            pltpu.SemaphoreType.DMA,
        ],
    )
    return f(dinvA, msvec, srcF, dstF, rawF)


SCH = 8
SNCH = RW // SCH
PW = 32


def _scatter_body(v_hbm, src_hbm, dst_hbm, norm_hbm, tok_hbm,
                  pout_hbm, tokout_hbm, sbufA, dbufA, nbufA, rows0, rows1, acc,
                  tokbuf, semg0, semg1, sems0, sems1):
    c = lax.axis_index("c")
    s = lax.axis_index("s")
    w = c * NS + s
    r0 = s * ROWS_PER_TILE
    eb = w * RW * 128
    rb = w * RW

    pltpu.sync_copy(tok_hbm, tokbuf)

    @pl.when(w == 0)
    def _():
        pltpu.sync_copy(tokbuf, tokout_hbm)

    pltpu.sync_copy(src_hbm.at[pl.ds(eb, RW * 128)], sbufA)
    pltpu.sync_copy(dst_hbm.at[pl.ds(rb, RW)], dbufA)
    pltpu.sync_copy(norm_hbm.at[pl.ds(eb, RW * 128)], nbufA)

    z16f = jnp.zeros((16,), _f32)

    def zrow(rr, _):
        for k in range(PW // 16):
            rows0[rr, pl.ds(k * 16, 16)] = z16f
        return 0
    lax.fori_loop(0, 128, zrow, 0)
    for zz in range(ROWS_PER_TILE // 128):
        pltpu.sync_copy(rows0.at[pl.ds(0, 128)],
                        acc.at[pl.ds(r0 + zz * 128, 128)])
    plsc.subcore_barrier()

    rows = (rows0, rows1)
    semg = (semg0, semg1)
    sems = (sems0, sems1)

    def fire_gather(cc, b):
        for jj in range(SCH):
            pltpu.async_copy(
                v_hbm.at[sbufA.at[pl.ds((cc * SCH + jj) * 128, 128)]],
                rows[b].at[pl.ds(jj * 128, 128)], semg[b])

    def drain_gather(cc, b):
        for jj in range(SCH):
            pltpu.make_async_copy(
                v_hbm.at[sbufA.at[pl.ds((cc * SCH + jj) * 128, 128)]],
                rows[b].at[pl.ds(jj * 128, 128)], semg[b]).wait()

    def fire_scatter(cc, b):
        for jj in range(SCH):
            pltpu.async_copy(
                rows[b].at[pl.ds(jj * 128, 128)],
                acc.at[dbufA.at[cc * SCH + jj]], sems[b], add=True)

    def drain_scatter(cc, b):
        for jj in range(SCH):
            pltpu.make_async_copy(
                rows[b].at[pl.ds(jj * 128, 128)],
                acc.at[dbufA.at[cc * SCH + jj]], sems[b]).wait()

    def scale(cc, b):
        cb = cc * SCH * 128

        def group(g, _2):
            for i in range(16):
                sp = plsc.load_gather(nbufA, [_zc16() + (cb + g * 16 + i)])
                le = g * 16 + i
                for k in range(PW // 16):
                    rows[b][le, pl.ds(k * 16, 16)] = (
                        rows[b][le, pl.ds(k * 16, 16)] * sp)
            return 0
        lax.fori_loop(0, SCH * 8, group, 0)

    def proc(cc, b, first):
        drain_gather(cc, b)
        scale(cc, b)
        fire_scatter(cc, b)
        if not first:
            drain_scatter(cc - 1, 1 - b)
        nxt = jnp.where(cc + 1 < SNCH, cc + 1, 0)
        fire_gather(nxt, 1 - b)

    fire_gather(0, 0)
    proc(jnp.int32(0), 0, True)

    def step(cc2, _):
        proc(cc2 * 2 + 1, 1, False)
        proc(cc2 * 2 + 2, 0, False)
        return 0
    lax.fori_loop(0, (SNCH - 2) // 2, step, 0)

    proc(jnp.int32(SNCH - 1), 1, False)
    drain_scatter(SNCH - 1, 1)
    drain_gather(0, 0)

    plsc.subcore_barrier()
    pltpu.sync_copy(acc.at[pl.ds(r0, ROWS_PER_TILE)],
                    pout_hbm.at[c, pl.ds(r0, ROWS_PER_TILE)])


def _sc_scatter(v, srcF, dstR, normF, tok):
    mesh = plsc.VectorSubcoreMesh(core_axis_name="c", subcore_axis_name="s")
    f = pl.kernel(
        _scatter_body,
        out_type=[pltpu.HBM((NC, N_PAD, PW), _f32),
                  pltpu.HBM((16,), _f32)],
        mesh=mesh,
        compiler_params=pltpu.CompilerParams(needs_layout_passes=False, use_tc_tiling_on_sc=False),
        scratch_types=[
            pltpu.VMEM((RW * 128,), _i32),
            pltpu.VMEM((RW, 128), _i32),
            pltpu.VMEM((RW * 128,), _f32),
            pltpu.VMEM((SCH * 128, PW), _f32),
            pltpu.VMEM((SCH * 128, PW), _f32),
            pltpu.VMEM_SHARED((N_PAD, PW), _f32),
            pltpu.VMEM((16,), _f32),
            pltpu.SemaphoreType.DMA,
            pltpu.SemaphoreType.DMA,
            pltpu.SemaphoreType.DMA,
            pltpu.SemaphoreType.DMA,
        ],
    )
    return f(v, srcF, dstR, normF, tok)


def _combine_body(p_ref, v_ref, sn_ref, o_ref):
    sn = sn_ref[...][:, 0:1]
    o_ref[...] = p_ref[0] + p_ref[1] + sn * v_ref[...]


def _tc_combine(p, v, sn8):
    blk = 1280
    return pl.pallas_call(
        _combine_body,
        grid=(N_PAD // blk,),
        in_specs=[
            pl.BlockSpec((NC, blk, PW), lambda i: (0, i, 0)),
            pl.BlockSpec((blk, PW), lambda i: (i, 0)),
            pl.BlockSpec((blk, 8), lambda i: (i, 0)),
        ],
        out_specs=pl.BlockSpec((blk, PW), lambda i: (i, 0)),
        out_shape=jax.ShapeDtypeStruct((N_PAD, PW), _f32),
    )(p, v, sn8)


def _stage_e_body(p0_ref, c1_ref, c2_ref, c3_ref, c4_ref, b0_ref,
                  w1cat_ref, w10_ref, b10_ref, r_ref, r0_ref):
    b0 = b0_ref[...]
    h1 = jnp.concatenate([
        jnp.maximum(p0_ref[...], 0.0),
        jnp.maximum(c1_ref[...] + b0[1:2, :], 0.0),
        jnp.maximum(c2_ref[...] + b0[2:3, :], 0.0),
        jnp.maximum(c3_ref[...] + b0[3:4, :], 0.0),
        jnp.maximum(c4_ref[...] + b0[4:5, :], 0.0),
    ], axis=1)
    r_ref[...] = jnp.dot(h1, w1cat_ref[...], preferred_element_type=_f32)
    r0_ref[...] = (jnp.dot(h1, w10_ref[...], preferred_element_type=_f32)
                   + b10_ref[...])


def _stage_e(p0, c1, c2, c3, c4, b0p, W1cat, W10, b10):
    blk = 640
    full = lambda s: pl.BlockSpec(s, lambda i: (0, 0))
    row = lambda d: pl.BlockSpec((blk, d), lambda i: (i, 0))
    return pl.pallas_call(
        _stage_e_body,
        grid=(N_PAD // blk,),
        in_specs=[row(64), row(64), row(64), row(64), row(64),
                  full((8, 64)),
                  full((320, 256)), full((320, 64)), full((1, 64))],
        out_specs=[row(256), row(64)],
        out_shape=[
            jax.ShapeDtypeStruct((N_PAD, 256), _f32),
            jax.ShapeDtypeStruct((N_PAD, 64), _f32),
        ],
    )(p0, c1, c2, c3, c4, b0p, W1cat, W10, b10)


def _stage_f_body(r0_ref, d1_ref, d2_ref, d3_ref, d4_ref, b1_ref,
                  wf_ref, bf_ref, o_ref):
    b1 = b1_ref[...]
    h2 = jnp.concatenate([
        jnp.maximum(r0_ref[...], 0.0),
        jnp.maximum(d1_ref[...] + b1[1:2, :], 0.0),
        jnp.maximum(d2_ref[...] + b1[2:3, :], 0.0),
        jnp.maximum(d3_ref[...] + b1[3:4, :], 0.0),
        jnp.maximum(d4_ref[...] + b1[4:5, :], 0.0),
    ], axis=1)
    o_ref[...] = (jnp.dot(h2, wf_ref[...], preferred_element_type=_f32)
                  + bf_ref[...])


def _stage_f(r0, d1, d2, d3, d4, b1p, Wf, bf):
    blk = 640
    full = lambda s: pl.BlockSpec(s, lambda i: (0, 0))
    row = lambda d: pl.BlockSpec((blk, d), lambda i: (i, 0))
    return pl.pallas_call(
        _stage_f_body,
        grid=(N_PAD // blk,),
        in_specs=[row(64), row(64), row(64), row(64), row(64),
                  full((8, 64)), full((320, OUT)), full((1, OUT))],
        out_specs=row(OUT),
        out_shape=jax.ShapeDtypeStruct((N_PAD, OUT), _f32),
    )(r0, d1, d2, d3, d4, b1p, Wf, bf)


def kernel(x, edge_index, lins0_W, lins0_b, lins1_W, lins1_b, Wf, bf,
           Wp1, bp1, Wp2, bp2, Wp3, bp3, parsing0):
    src = edge_index[0]
    dst = edge_index[1]
    pad = E_PAD - E
    srcF = jnp.concatenate([src, jnp.zeros((pad,), _i32)])
    dstF = jnp.concatenate([dst, jnp.full((pad,), DUMP, _i32)])
    dstR = dstF.reshape(EROWS, 128)
    xP = jnp.zeros((N_PAD, D_IN), _f32).at[:N].set(x)

    P16 = jnp.zeros((16, 16), _f32).at[:OUT, :OUT].set(
        jax.nn.relu(SCALING * parsing0))
    W0cat = jnp.concatenate([lins0_W[j] for j in range(1, NPOW)], axis=1)
    W1cat = jnp.concatenate([lins1_W[j] for j in range(1, NPOW)], axis=1)
    b0p = jnp.zeros((8, 64), _f32).at[:NPOW].set(lins0_b)
    b1p = jnp.zeros((8, 64), _f32).at[:NPOW].set(lins1_b)
    Wp3p = jnp.zeros((64, 128), _f32).at[:, :OUT].set(Wp3)
    bp3p = jnp.zeros((128,), _f32).at[:OUT].set(bp3)

    At, Bt, q, p0 = _stage_a(xP, Wp1, bp1[None], Wp2, bp2[None],
                             Wp3p, bp3p[None], P16, W0cat,
                             lins0_W[0], lins0_b[0][None])

    rawF, statsP, degP = _stage_b(At, Bt, srcF, dstR)
    msarr, dinv8, sn8 = _stage_c2(statsP, degP)
    dinvA = dinv8[:, 0]
    msvec = msarr[0, :16]
    normF = _stage_c3(dinvA, msvec, srcF, dstF, rawF)

    def S(v, tok):
        p, tok2 = _sc_scatter(v, srcF, dstR, normF, tok)
        return _tc_combine(p, v, sn8), tok2

    def run_chains(chains, tok):
        outs = []
        cur = list(chains)
        for step in range(len(cur)):
            for j in range(step, len(cur)):
                cur[j], tok = S(cur[j], tok)
            outs.append(cur[step])
        return outs, tok

    def run_layer(m, tok):
        halves = [m[:, 32 * t:32 * (t + 1)] for t in range(8)]
        outs = [None] * 8
        for step in range(4):
            for t in range(2 * step, 8):
                halves[t], tok = S(halves[t], tok)
            outs[2 * step] = halves[2 * step]
            outs[2 * step + 1] = halves[2 * step + 1]
        full = [jnp.concatenate([outs[2 * jj], outs[2 * jj + 1]], axis=1)
                for jj in range(4)]
        return full, tok

    tok = msvec
    (c1, c2, c3, c4), tok = run_layer(q, tok)
    r, r0 = _stage_e(p0, c1, c2, c3, c4, b0p, W1cat, lins1_W[0],
                     lins1_b[0][None])
    (d1, d2, d3, d4), tok = run_layer(r, tok)
    out = _stage_f(r0, d1, d2, d3, d4, b1p, Wf, bf[None])
    return out[:N]
